# Initial kernel scaffold; baseline (speedup 1.0000x reference)
import jax
import jax.numpy as jnp
from jax.experimental import pallas as pl


def kernel(x, edge_index, W1, b1, a1, W2, b2, a2):
    raise NotImplementedError("write your pallas kernel here")


# speedup vs baseline: 19.8919x; 19.8919x over previous
import dataclasses
import functools

import jax
import jax.numpy as jnp
from jax import lax
from jax.experimental import pallas as pl
from jax.experimental.pallas import tpu as pltpu
from jax.experimental.pallas import tpu_sc as plsc

N = 10000
E = 320000
D_IN = 128
HID = 64
N_CLS = 40

NC, NS, L = 2, 16, 16
NW = NC * NS
CHUNK = 1024
NCHUNK = 11
T_EDGES = CHUNK * NCHUNK
E_TOT = E + N
E_PAD = NW * T_EDGES
N_TAB = N + 112
ROWS_PT = N_TAB // NS

W1_TAB = 80
ONES1 = HID
W2_TAB = 64
ONES2 = 48


def _sc_edge_kernel_body(with_cnt, d_tab, as_col,
                         hext, adt, row2, col2, ztab, zcnt, ones_hbm,
                         u_out, cnt_out,
                         row_v, col_v, hrows0, hrows1, adr0, adr1, p_buf,
                         ones_v, u_sh, cnt_sh, sems):
    cid = lax.axis_index("c")
    sid = lax.axis_index("s")
    wid = cid * NS + sid
    nvec = d_tab // L

    pltpu.sync_copy(ztab.at[pl.ds(sid * ROWS_PT, ROWS_PT)],
                    u_sh.at[pl.ds(sid * ROWS_PT, ROWS_PT)])
    if with_cnt:
        pltpu.sync_copy(zcnt.at[pl.ds(sid * ROWS_PT, ROWS_PT)],
                        cnt_sh.at[pl.ds(sid * ROWS_PT, ROWS_PT)])
        pltpu.sync_copy(ones_hbm, ones_v)
    plsc.subcore_barrier()

    base128 = wid * (T_EDGES // 128)
    hbufs = (hrows0, hrows1)
    abufs = (adr0, adr1)

    def fire(q):
        return (pltpu.async_copy(hext.at[row_v.at[q]], hbufs[q % 2],
                                 sems.at[(q % 2) * 2]),
                pltpu.async_copy(adt.at[col_v.at[q]], abufs[q % 2],
                                 sems.at[(q % 2) * 2 + 1]))

    @pl.loop(0, NCHUNK)
    def _chunk(k):
        r0 = base128 + k * 8
        pltpu.sync_copy(row2.at[pl.ds(r0, 8)], row_v)
        pltpu.sync_copy(col2.at[pl.ds(r0, 8)], col_v)
        i16 = jax.lax.iota(jnp.int32, 16)
        i16z = jnp.zeros((L,), jnp.int32)
        c_as = i16z + as_col
        cps = {0: fire(0)}
        for q in range(8):
            hbuf, abuf = hbufs[q % 2], abufs[q % 2]
            if q + 1 < 8:
                cps[q + 1] = fire(q + 1)
            cps[q][0].wait()
            cps[q][1].wait()
            for t in range(8):
                jv = i16 + t * L
                s = (plsc.load_gather(hbuf, [jv, c_as])
                     + plsc.load_gather(abuf, [jv, i16z]))
                e = jnp.where(s > 0, s, 0.2 * s)
                p_buf[pl.ds(t * L, L)] = jnp.exp(e)
            @pl.loop(0, 128, step=4)
            def _scale(i):
                for u in range(4):
                    j = i + u
                    pb = plsc.load_gather(
                        p_buf, [jnp.full((L,), 0, jnp.int32) + j])
                    for c in range(nvec):
                        hbuf[j, pl.ds(c * L, L)] = (
                            hbuf[j, pl.ds(c * L, L)] * pb)
            pltpu.sync_copy(hbuf, u_sh.at[col_v.at[q]], add=True)
            if with_cnt:
                pltpu.sync_copy(ones_v, cnt_sh.at[col_v.at[q]], add=True)

    plsc.subcore_barrier()
    pltpu.sync_copy(u_sh.at[pl.ds(sid * ROWS_PT, ROWS_PT)],
                    u_out.at[cid, pl.ds(sid * ROWS_PT, ROWS_PT)])
    if with_cnt:
        pltpu.sync_copy(cnt_sh.at[pl.ds(sid * ROWS_PT, ROWS_PT)],
                        cnt_out.at[cid, pl.ds(sid * ROWS_PT, ROWS_PT)])


def _sc_body_nocnt(d_tab, as_col, hext, adt, row2, col2, ztab, zcnt,
                   ones_hbm, u_out, *scratch):
    _sc_edge_kernel_body(False, d_tab, as_col, hext, adt, row2, col2, ztab,
                         zcnt, ones_hbm, u_out, None, *scratch)


def _sc_compiler_params():
    return pltpu.CompilerParams(needs_layout_passes=False,
                                use_tc_tiling_on_sc=False)


RB = 1000
GRID = N // RB


def _prep1_body(x_ref, w1t_ref, b1_ref, ac_ref, hext_ref, adt_ref):
    h = jnp.dot(x_ref[...], w1t_ref[...],
                preferred_element_type=jnp.float32) + b1_ref[...]
    proj = lax.dot_general(h, ac_ref[...], (((1,), (1,)), ((), ())),
                           preferred_element_type=jnp.float32)
    ones = jnp.ones((RB, 1), jnp.float32)
    zpad = jnp.zeros((RB, W1_TAB - HID - 2), jnp.float32)
    hext_ref[...] = jnp.concatenate([h, ones, proj[:, 0:1], zpad], axis=1)
    adt_ref[...] = jnp.concatenate(
        [proj[:, 1:2], jnp.zeros((RB, L - 1), jnp.float32)], axis=1)


def _prep2_body(u_ref, cnt_ref, w2t_ref, b2_ref, ac_ref, hext_ref, adt_ref):
    usum = u_ref[0] + u_ref[1]
    cnt = cnt_ref[0, :, 0:1] + cnt_ref[1, :, 0:1]
    denom = usum[:, ONES1:ONES1 + 1]
    h1 = usum[:, :HID] / (denom * jnp.maximum(cnt, 1.0))
    z = jnp.dot(h1, w2t_ref[...], preferred_element_type=jnp.float32) + b2_ref[...]
    proj = lax.dot_general(z, ac_ref[...], (((1,), (1,)), ((), ())),
                           preferred_element_type=jnp.float32)
    zpad1 = jnp.zeros((RB, ONES2 - N_CLS), jnp.float32)
    ones = jnp.ones((RB, 1), jnp.float32)
    zpad2 = jnp.zeros((RB, W2_TAB - ONES2 - 2), jnp.float32)
    hext_ref[...] = jnp.concatenate([z, zpad1, ones, proj[:, 0:1], zpad2],
                                    axis=1)
    adt_ref[...] = jnp.concatenate(
        [proj[:, 1:2], jnp.zeros((RB, L - 1), jnp.float32)], axis=1)


def _final_body(u_ref, cnt_ref, out_ref):
    usum = u_ref[0] + u_ref[1]
    cnt = cnt_ref[0, :, 0:1] + cnt_ref[1, :, 0:1]
    denom = usum[:, ONES2:ONES2 + 1]
    o = usum[:, :N_CLS] / (denom * jnp.maximum(cnt, 1.0))
    m = jnp.max(o, axis=1, keepdims=True)
    ex = jnp.exp(o - m)
    lse = jnp.log(jnp.sum(ex, axis=1, keepdims=True))
    out_ref[...] = o - m - lse


def _prep1(x, w1t, b1, ac):
    return pl.pallas_call(
        _prep1_body,
        grid=(GRID,),
        in_specs=[
            pl.BlockSpec((RB, D_IN), lambda i: (i, 0)),
            pl.BlockSpec((D_IN, HID), lambda i: (0, 0)),
            pl.BlockSpec((1, HID), lambda i: (0, 0)),
            pl.BlockSpec((2, HID), lambda i: (0, 0)),
        ],
        out_specs=[
            pl.BlockSpec((RB, W1_TAB), lambda i: (i, 0)),
            pl.BlockSpec((RB, L), lambda i: (i, 0)),
        ],
        out_shape=[
            jax.ShapeDtypeStruct((N, W1_TAB), jnp.float32),
            jax.ShapeDtypeStruct((N, L), jnp.float32),
        ],
    )(x, w1t, b1, ac)


def _prep2(u1, cnt, w2t, b2, ac):
    return pl.pallas_call(
        _prep2_body,
        grid=(GRID,),
        in_specs=[
            pl.BlockSpec((NC, RB, W1_TAB), lambda i: (0, i, 0)),
            pl.BlockSpec((NC, RB, L), lambda i: (0, i, 0)),
            pl.BlockSpec((HID, N_CLS), lambda i: (0, 0)),
            pl.BlockSpec((1, N_CLS), lambda i: (0, 0)),
            pl.BlockSpec((2, N_CLS), lambda i: (0, 0)),
        ],
        out_specs=[
            pl.BlockSpec((RB, W2_TAB), lambda i: (i, 0)),
            pl.BlockSpec((RB, L), lambda i: (i, 0)),
        ],
        out_shape=[
            jax.ShapeDtypeStruct((N, W2_TAB), jnp.float32),
            jax.ShapeDtypeStruct((N, L), jnp.float32),
        ],
    )(u1, cnt, w2t, b2, ac)


def _final(u2, cnt):
    return pl.pallas_call(
        _final_body,
        grid=(GRID,),
        in_specs=[
            pl.BlockSpec((NC, RB, W2_TAB), lambda i: (0, i, 0)),
            pl.BlockSpec((NC, RB, L), lambda i: (0, i, 0)),
        ],
        out_specs=pl.BlockSpec((RB, N_CLS), lambda i: (i, 0)),
        out_shape=jax.ShapeDtypeStruct((N, N_CLS), jnp.float32),
    )(u2, cnt)


def _sc_layer1(hext, adt, row2, col2):
    mesh = plsc.VectorSubcoreMesh(core_axis_name="c", subcore_axis_name="s")
    ztab = jnp.zeros((N_TAB, W1_TAB), jnp.float32)
    zcnt = jnp.zeros((N_TAB, L), jnp.float32)
    ones = jnp.zeros((128, L), jnp.float32).at[:, 0].set(1.0)
    k = pl.kernel(
        functools.partial(_sc_edge_kernel_body, True, W1_TAB, ONES1 + 1),
        out_type=(jax.ShapeDtypeStruct((NC, N_TAB, W1_TAB), jnp.float32),
                  jax.ShapeDtypeStruct((NC, N_TAB, L), jnp.float32)),
        mesh=mesh,
        compiler_params=_sc_compiler_params(),
        scratch_types=[
            pltpu.VMEM((8, 128), jnp.int32),
            pltpu.VMEM((8, 128), jnp.int32),
            pltpu.VMEM((128, W1_TAB), jnp.float32),
            pltpu.VMEM((128, W1_TAB), jnp.float32),
            pltpu.VMEM((128, L), jnp.float32),
            pltpu.VMEM((128, L), jnp.float32),
            pltpu.VMEM((128,), jnp.float32),
            pltpu.VMEM((128, L), jnp.float32),
            pltpu.VMEM_SHARED((N_TAB, W1_TAB), jnp.float32),
            pltpu.VMEM_SHARED((N_TAB, L), jnp.float32),
            pltpu.SemaphoreType.DMA((2,)),
        ],
    )
    return k(hext, adt, row2, col2, ztab, zcnt, ones)


def _sc_layer2(hext, adt, row2, col2):
    mesh = plsc.VectorSubcoreMesh(core_axis_name="c", subcore_axis_name="s")
    ztab = jnp.zeros((N_TAB, W2_TAB), jnp.float32)
    zcnt = jnp.zeros((N_TAB, L), jnp.float32)
    ones = jnp.zeros((128, L), jnp.float32).at[:, 0].set(1.0)
    k = pl.kernel(
        functools.partial(_sc_body_nocnt, W2_TAB, ONES2 + 1),
        out_type=jax.ShapeDtypeStruct((NC, N_TAB, W2_TAB), jnp.float32),
        mesh=mesh,
        compiler_params=_sc_compiler_params(),
        scratch_types=[
            pltpu.VMEM((8, 128), jnp.int32),
            pltpu.VMEM((8, 128), jnp.int32),
            pltpu.VMEM((128, W2_TAB), jnp.float32),
            pltpu.VMEM((128, W2_TAB), jnp.float32),
            pltpu.VMEM((128, L), jnp.float32),
            pltpu.VMEM((128, L), jnp.float32),
            pltpu.VMEM((128,), jnp.float32),
            pltpu.VMEM((128, L), jnp.float32),
            pltpu.VMEM_SHARED((N_TAB, W2_TAB), jnp.float32),
            pltpu.VMEM_SHARED((N_TAB, L), jnp.float32),
            pltpu.SemaphoreType.DMA((2,)),
        ],
    )
    return k(hext, adt, row2, col2, ztab, zcnt, ones)


def kernel(x, edge_index, W1, b1, a1, W2, b2, a2):
    sl = jnp.arange(N, dtype=jnp.int32)
    npad = E_PAD - E_TOT
    pad_row = (jnp.arange(npad, dtype=jnp.int32) * 7919) % N
    pad_col = N + (jnp.arange(npad, dtype=jnp.int32) % 112)
    rowp = jnp.concatenate([edge_index[0], sl, pad_row]).reshape(-1, 128)
    colp = jnp.concatenate([edge_index[1], sl, pad_col]).reshape(-1, 128)

    w1t = W1.T
    b1r = b1.reshape(1, HID)
    ac1 = jnp.stack([a1[:HID, 0], a1[HID:, 0]])
    w2t = W2.T
    b2r = b2.reshape(1, N_CLS)
    ac2 = jnp.stack([a2[:N_CLS, 0], a2[N_CLS:, 0]])

    hext1, adt1 = _prep1(x, w1t, b1r, ac1)
    u1, cnt = _sc_layer1(hext1, adt1, rowp, colp)
    hext2, adt2 = _prep2(u1, cnt, w2t, b2r, ac2)
    u2 = _sc_layer2(hext2, adt2, rowp, colp)
    return _final(u2, cnt)
